# baseline (speedup 1.0000x reference)
import functools

import jax
import jax.numpy as jnp
from jax import lax
from jax.experimental import pallas as pl
from jax.experimental.pallas import tpu as pltpu
from jax.experimental.pallas import tpu_sc as plsc

_D = 65536
_d = 1024
_NC = 2
_NS = 16
_NW = _NC * _NS
_RW = _D // _NW
_CH = 32
_NCH = _RW // _CH
_CHW = _CH * _d
_JU = 4


def _sc_body(z_hbm, p_hbm, t_hbm, out_hbm, z_v, pa_v, pb_v, o_v, t_v, s_v, sem_a, sem_b):
    wid = lax.axis_index("s") * _NC + lax.axis_index("c")
    row0 = wid * _RW
    elem0 = row0 * _d

    pltpu.sync_copy(z_hbm, z_v)
    pltpu.sync_copy(t_hbm.at[pl.ds(row0, _RW)], t_v)

    pltpu.async_copy(p_hbm.at[pl.ds(elem0, _CHW)], pa_v, sem_a)
    pltpu.async_copy(p_hbm.at[pl.ds(elem0 + _CHW, _CHW)], pb_v, sem_b)

    def wait_chunk(buf, sem):
        pltpu.make_async_copy(p_hbm.at[pl.ds(0, _CHW)], buf, sem).wait()

    def compute_chunk(p_v, g):
        def grp_body(gi, _):
            rbase = gi * (16 * _d)

            def j_body(jj, accs):
                accs = list(accs)
                for ju in range(_JU):
                    j = jj * _JU + ju
                    zj = z_v[pl.ds(j * 16, 16)]
                    for r in range(16):
                        pv = p_v[pl.ds(rbase + r * _d + j * 16, 16)]
                        accs[r] = accs[r] + pv * zj
                return tuple(accs)

            accs = lax.fori_loop(
                0, (_d // 16) // _JU, j_body,
                tuple(jnp.zeros((16,), jnp.float32) for _ in range(16)),
            )
            off = g * _CH + gi * 16
            for r in range(16):
                base = 32 * r
                a = accs[r]
                s_v[pl.ds(base, 16)] = a
                a = a + s_v[pl.ds(base + 8, 16)]
                s_v[pl.ds(base, 16)] = a
                a = a + s_v[pl.ds(base + 4, 16)]
                s_v[pl.ds(base, 16)] = a
                a = a + s_v[pl.ds(base + 2, 16)]
                s_v[pl.ds(base, 16)] = a
                a = a + s_v[pl.ds(base + 1, 16)]
                o_v[pl.ds(off + r, 16)] = a
            return 0

        lax.fori_loop(0, _CH // 16, grp_body, 0)

    def pair_body(k, _):
        g_a = 2 * k
        wait_chunk(pa_v, sem_a)
        compute_chunk(pa_v, g_a)

        @pl.when(g_a + 2 < _NCH)
        def _():
            pltpu.async_copy(
                p_hbm.at[pl.ds(elem0 + (g_a + 2) * _CHW, _CHW)], pa_v, sem_a)

        wait_chunk(pb_v, sem_b)
        compute_chunk(pb_v, g_a + 1)

        @pl.when(g_a + 3 < _NCH)
        def _():
            pltpu.async_copy(
                p_hbm.at[pl.ds(elem0 + (g_a + 3) * _CHW, _CHW)], pb_v, sem_b)

        return 0

    lax.fori_loop(0, _NCH // 2, pair_body, 0)

    def theta_body(k, _):
        sl = pl.ds(k * 16, 16)
        o_v[sl] = o_v[sl] + t_v[sl]
        return 0

    lax.fori_loop(0, _RW // 16, theta_body, 0)

    pltpu.sync_copy(o_v.at[pl.ds(0, _RW)], out_hbm.at[pl.ds(row0, _RW)])


_sc_call = functools.partial(
    pl.kernel,
    out_type=jax.ShapeDtypeStruct((_D,), jnp.float32),
    mesh=plsc.VectorSubcoreMesh(core_axis_name="c", subcore_axis_name="s"),
    scratch_types=[
        pltpu.VMEM((_d,), jnp.float32),
        pltpu.VMEM((_CHW,), jnp.float32),
        pltpu.VMEM((_CHW,), jnp.float32),
        pltpu.VMEM((_RW + 16,), jnp.float32),
        pltpu.VMEM((_RW,), jnp.float32),
        pltpu.VMEM((512,), jnp.float32),
        pltpu.SemaphoreType.DMA,
        pltpu.SemaphoreType.DMA,
    ],
)(_sc_body)


def kernel(z, P, theta_base):
    return _sc_call(z, P.reshape(_D * _d), theta_base)


# speedup vs baseline: 4.0198x; 4.0198x over previous
import jax
import jax.numpy as jnp
from jax.experimental import pallas as pl


_D = 65536
_d = 1024
_BLK = 2048


def _matvec_body(p_ref, z_ref, t_ref, o_ref):
    acc = jnp.sum(p_ref[...] * z_ref[...], axis=1)
    o_ref[...] = t_ref[...] + acc


def kernel(z, P, theta_base):
    D, d = P.shape
    zb = z.reshape(1, d)
    out = pl.pallas_call(
        _matvec_body,
        grid=(D // _BLK,),
        in_specs=[
            pl.BlockSpec((_BLK, d), lambda i: (i, 0)),
            pl.BlockSpec((1, d), lambda i: (0, 0)),
            pl.BlockSpec((_BLK,), lambda i: (i,)),
        ],
        out_specs=pl.BlockSpec((_BLK,), lambda i: (i,)),
        out_shape=jax.ShapeDtypeStruct((D,), jnp.float32),
    )(P, zb, theta_base)
    return out
